# baseline (speedup 1.0000x reference)
import jax
import jax.numpy as jnp
from jax.experimental import pallas as pl
from jax.experimental.pallas import tpu as pltpu

_NCHUNKS = 32


def _copy_body(x_ref, o_ref, *sems):
    rows = x_ref.shape[0]
    base_chunk = rows // _NCHUNKS
    copies = []
    start = 0
    for i in range(_NCHUNKS):
        n = base_chunk + (1 if i < rows - base_chunk * _NCHUNKS else 0)
        cp = pltpu.make_async_copy(
            x_ref.at[pl.ds(start, n)], o_ref.at[pl.ds(start, n)], sems[i]
        )
        cp.start()
        copies.append(cp)
        start += n
    for cp in copies:
        cp.wait()


def kernel(src_tokens, table):
    del table
    return pl.pallas_call(
        _copy_body,
        out_shape=jax.ShapeDtypeStruct(src_tokens.shape, src_tokens.dtype),
        in_specs=[pl.BlockSpec(memory_space=pl.ANY)],
        out_specs=pl.BlockSpec(memory_space=pl.ANY),
        scratch_shapes=[pltpu.SemaphoreType.DMA] * _NCHUNKS,
    )(src_tokens)


# speedup vs baseline: 7.7381x; 7.7381x over previous
import jax
import jax.numpy as jnp
from jax.experimental import pallas as pl
from jax.experimental.pallas import tpu as pltpu

_GRID = 8


def _copy_body(x_ref, o_ref):
    o_ref[...] = x_ref[...]


def kernel(src_tokens, table):
    del table
    B, L = src_tokens.shape
    rows = B // _GRID
    return pl.pallas_call(
        _copy_body,
        out_shape=jax.ShapeDtypeStruct((B, L), src_tokens.dtype),
        grid=(_GRID,),
        in_specs=[pl.BlockSpec((rows, L), lambda i: (i, 0))],
        out_specs=pl.BlockSpec((rows, L), lambda i: (i, 0)),
        compiler_params=pltpu.CompilerParams(
            dimension_semantics=("arbitrary",),
        ),
    )(src_tokens)
